# baseline (speedup 1.0000x reference)
import functools

import jax
import jax.numpy as jnp
from jax import lax
from jax.experimental import pallas as pl
from jax.experimental.pallas import tpu as pltpu
from jax.experimental.pallas import tpu_sc as plsc

_NUM_CLASS = 100
_FEAT = 128
_N = 320000
_NC = 2
_NS = 16
_NW = _NC * _NS
_ROWS_PER_W = _N // _NW
_SCHUNK = 256
_CHUNK = 128
_NSUP = _ROWS_PER_W // _SCHUNK
_TAIL = _ROWS_PER_W - _NSUP * _SCHUNK
_ACLASS = 112


def _sc_segment_sum(batch_hbm, tgt_hbm, parts_hbm,
                    rows_buf0, rows_buf1, tgt_buf00, tgt_buf01,
                    tgt_buf10, tgt_buf11, rows_tail, tgt_tail, zero_buf, acc,
                    gsem0, gsem1, zsem):
    cid = lax.axis_index("c")
    sid = lax.axis_index("s")
    wid = cid * _NS + sid
    base = wid * _ROWS_PER_W
    abase = sid * _ACLASS
    rows_bufs = (rows_buf0, rows_buf1)
    tgt_bufs = ((tgt_buf00, tgt_buf01), (tgt_buf10, tgt_buf11))
    gsems = (gsem0, gsem1)

    zeros16 = jnp.zeros((16,), jnp.float32)

    def zero_body(i, carry):
        r = i // (_FEAT // 16)
        c = i % (_FEAT // 16)
        zero_buf[r, pl.ds(c * 16, 16)] = zeros16
        return carry

    lax.fori_loop(0, 8 * (_FEAT // 16), zero_body, 0)
    for k in range(_ACLASS // 8):
        pltpu.async_copy(zero_buf, acc.at[pl.ds(abase + k * 8, 8)], zsem)
    for k in range(_ACLASS // 8):
        pltpu.make_async_copy(zero_buf, acc.at[pl.ds(abase + k * 8, 8)],
                              zsem).wait()

    def bias_tgts(tb):
        bias = jnp.zeros((16,), jnp.int32) + abase
        for k in range(_CHUNK // 16):
            tb[pl.ds(k * 16, 16)] = tb[pl.ds(k * 16, 16)] + bias

    def start_gather(si, b):
        off = base + si * _SCHUNK
        pltpu.async_copy(tgt_hbm.at[pl.ds(off, _CHUNK)], tgt_bufs[b][0],
                         gsems[b])
        pltpu.async_copy(tgt_hbm.at[pl.ds(off + _CHUNK, _CHUNK)],
                         tgt_bufs[b][1], gsems[b])
        pltpu.async_copy(batch_hbm.at[pl.ds(off, _SCHUNK)], rows_bufs[b],
                         gsems[b])

    def wait_gather(si, b):
        off = base + si * _SCHUNK
        pltpu.make_async_copy(tgt_hbm.at[pl.ds(off, _CHUNK)], tgt_bufs[b][0],
                              gsems[b]).wait()
        pltpu.make_async_copy(tgt_hbm.at[pl.ds(off + _CHUNK, _CHUNK)],
                              tgt_bufs[b][1], gsems[b]).wait()
        pltpu.make_async_copy(batch_hbm.at[pl.ds(off, _SCHUNK)], rows_bufs[b],
                              gsems[b]).wait()

    start_gather(0, 0)
    start_gather(1, 1)

    def super_body(i, carry):
        for b in range(2):
            si = 2 * i + b

            @pl.when(si < _NSUP)
            def _():
                wait_gather(si, b)
                bias_tgts(tgt_bufs[b][0])
                bias_tgts(tgt_bufs[b][1])
                pltpu.sync_copy(rows_bufs[b].at[pl.ds(0, _CHUNK)],
                                acc.at[tgt_bufs[b][0]], add=True)
                pltpu.sync_copy(rows_bufs[b].at[pl.ds(_CHUNK, _CHUNK)],
                                acc.at[tgt_bufs[b][1]], add=True)

                @pl.when(si + 2 < _NSUP)
                def _():
                    start_gather(si + 2, b)
        return carry

    lax.fori_loop(0, (_NSUP + 1) // 2, super_body, 0)

    off = base + _NSUP * _SCHUNK
    pltpu.sync_copy(tgt_hbm.at[pl.ds(off, _TAIL)], tgt_tail)
    pltpu.sync_copy(batch_hbm.at[pl.ds(off, _TAIL)], rows_tail)
    tgt_tail[...] = tgt_tail[...] + (jnp.zeros((16,), jnp.int32) + abase)
    pltpu.sync_copy(rows_tail, acc.at[tgt_tail], add=True)

    pltpu.sync_copy(acc.at[pl.ds(abase, _ACLASS)], parts_hbm.at[wid])


_sc_call = functools.partial(
    pl.kernel,
    out_type=jax.ShapeDtypeStruct((_NW, _ACLASS, _FEAT), jnp.float32),
    mesh=plsc.VectorSubcoreMesh(core_axis_name="c", subcore_axis_name="s"),
    scratch_types=(
        [pltpu.VMEM((_SCHUNK, _FEAT), jnp.float32)] * 2
        + [pltpu.VMEM((_CHUNK,), jnp.int32)] * 4
        + [
            pltpu.VMEM((_TAIL, _FEAT), jnp.float32),
            pltpu.VMEM((_TAIL,), jnp.int32),
            pltpu.VMEM((8, _FEAT), jnp.float32),
            pltpu.VMEM_SHARED((_NS * _ACLASS, _FEAT), jnp.float32),
        ]
        + [pltpu.SemaphoreType.DMA] * 3
    ),
)(_sc_segment_sum)


def _combine(parts_ref, cs_ref, o_ref):
    o_ref[...] = cs_ref[...] + jnp.sum(parts_ref[:, :_NUM_CLASS, :], axis=0)


def kernel(batch_samples, targets, idx, class_sums):
    del idx
    parts = _sc_call(batch_samples, targets)
    return pl.pallas_call(
        _combine,
        out_shape=jax.ShapeDtypeStruct((_NUM_CLASS, _FEAT), jnp.float32),
    )(parts, class_sums)
